# Initial kernel scaffold; baseline (speedup 1.0000x reference)
import jax
import jax.numpy as jnp
from jax.experimental import pallas as pl


def kernel(code_inputs, attn_mask, position_idx, New_DFG_ids, params):
    raise NotImplementedError("write your pallas kernel here")


# speedup vs baseline: 1.6322x; 1.6322x over previous
import functools
import math

import jax
import jax.numpy as jnp
from jax import lax
from jax.experimental import pallas as pl
from jax.experimental.pallas import tpu as pltpu
from jax.experimental.pallas import tpu_sc as plsc

_HID = 768
_L = 512
_BS = 4
_BSF = 10
_NH = 12
_HD = 64
_FF = 3072
_ALPA = 0.6
_DC = 64

_NW = 32
_WTOT = _BS * _L + _BS * _DC * _BSF
_WPW = _WTOT // _NW
_PTOT = _BS * _L
_PPW = _PTOT // _NW


def _sc_gather(wemb, pemb, widx, pidx):
    mesh = plsc.VectorSubcoreMesh(core_axis_name="c", subcore_axis_name="s")

    @functools.partial(
        pl.kernel,
        out_type=[
            jax.ShapeDtypeStruct((_WTOT, _HID), jnp.float32),
            jax.ShapeDtypeStruct((_PTOT, _HID), jnp.float32),
        ],
        mesh=mesh,
        scratch_types=[
            pltpu.VMEM((_WPW,), jnp.int32),
            pltpu.VMEM((_PPW,), jnp.int32),
            pltpu.VMEM((_WPW, _HID), jnp.float32),
            pltpu.SemaphoreType.DMA,
        ],
    )
    def k(wemb_h, pemb_h, widx_h, pidx_h, wout_h, pout_h,
          widx_v, pidx_v, rows_v, sem):
        wid = lax.axis_index("s") * 2 + lax.axis_index("c")
        wb = wid * _WPW
        pltpu.sync_copy(widx_h.at[pl.ds(wb, _WPW)], widx_v)
        pltpu.async_copy(wemb_h.at[widx_v], rows_v, sem).wait()
        pltpu.sync_copy(rows_v, wout_h.at[pl.ds(wb, _WPW)])
        pb = wid * _PPW
        pltpu.sync_copy(pidx_h.at[pl.ds(pb, _PPW)], pidx_v)
        pltpu.async_copy(pemb_h.at[pidx_v], rows_v.at[pl.ds(0, _PPW)], sem).wait()
        pltpu.sync_copy(rows_v.at[pl.ds(0, _PPW)], pout_h.at[pl.ds(pb, _PPW)])

    return k(wemb, pemb, widx, pidx)


def _gru_body(x_ref, wih_ref, whh_ref, bih_ref, bhh_ref, out_ref):
    n = x_ref.shape[0]
    wih = wih_ref[...]
    whh = whh_ref[...]
    bih = bih_ref[...]
    bhh = bhh_ref[...]
    h = jnp.zeros((n, _HID), jnp.float32)
    for t in range(_BSF):
        x = x_ref[:, t, :]
        gi = jnp.dot(x, wih, preferred_element_type=jnp.float32) + bih
        gh = jnp.dot(h, whh, preferred_element_type=jnp.float32) + bhh
        r = jax.nn.sigmoid(gi[:, :_HID] + gh[:, :_HID])
        z = jax.nn.sigmoid(gi[:, _HID:2 * _HID] + gh[:, _HID:2 * _HID])
        nn = jnp.tanh(gi[:, 2 * _HID:] + r * gh[:, 2 * _HID:])
        h = (1.0 - z) * nn + z * h
    out_ref[...] = h


def _run_gru(dfg_emb, wih_t, whh_t, bih, bhh):
    n = dfg_emb.shape[0]
    return pl.pallas_call(
        _gru_body,
        out_shape=jax.ShapeDtypeStruct((n, _HID), jnp.float32),
    )(dfg_emb, wih_t, whh_t, bih, bhh)


def _ln(x, g, b):
    m = jnp.mean(x, axis=-1, keepdims=True)
    v = jnp.mean((x - m) * (x - m), axis=-1, keepdims=True)
    return (x - m) / jnp.sqrt(v + 1e-5) * g + b


def _softmax(x):
    m = jnp.max(x, axis=-1, keepdims=True)
    e = jnp.exp(x - m)
    return e / jnp.sum(e, axis=-1, keepdims=True)


def _fused_body(pos_all_ref, emb_ref, prow_ref, g_ref, pos64_ref,
                qw_ref, kw_ref, vw_ref, fw_ref,
                lne_ref, wq_ref, wk_ref, wv_ref, wo_ref,
                ln1_ref, w1_ref, b1_ref, w2_ref,
                ln2_ref, pw_ref, bias_ref,
                out_ref, ctx_ref):
    b = pl.program_id(0)
    pos_all = pos_all_ref[...]
    dfg_len_all = jnp.sum((pos_all == 0).astype(jnp.int32), axis=1,
                          keepdims=True)
    dmax = jnp.max(dfg_len_all)
    pos_b = pos_all_ref[pl.ds(b, 1), :]
    token_f = (pos_b >= 2).astype(jnp.float32)
    didx = jnp.sum((pos_b >= 2).astype(jnp.int32))
    dlen = jnp.sum((pos_b == 0).astype(jnp.int32))

    biases = bias_ref[...]
    qb, kb, vb, fb = (biases[0:1, :_HID], biases[1:2, :_HID],
                      biases[2:3, :_HID], biases[3:4, :_HID])
    bq, bk, bv, bo = (biases[4:5, :_HID], biases[5:6, :_HID],
                      biases[6:7, :_HID], biases[7:8, :_HID])
    lne = lne_ref[...]
    ln1 = ln1_ref[...]
    ln2 = ln2_ref[...]

    dfg = g_ref[...] + pos64_ref[...]
    q = jnp.dot(dfg, qw_ref[...], preferred_element_type=jnp.float32) + qb
    k = jnp.dot(dfg, kw_ref[...], preferred_element_type=jnp.float32) + kb
    v = jnp.dot(dfg, vw_ref[...], preferred_element_type=jnp.float32) + vb
    sc = lax.dot_general(q, k, (((1,), (1,)), ((), ())),
                         preferred_element_type=jnp.float32)
    sc = sc * (1.0 / math.sqrt(_HID))
    col = lax.broadcasted_iota(jnp.int32, (_DC, _DC), 1)
    sc = jnp.where(col < dmax, sc, -jnp.inf)
    dfgo = jnp.dot(_softmax(sc), v, preferred_element_type=jnp.float32)
    dfgo = jnp.dot(dfgo, fw_ref[...], preferred_element_type=jnp.float32) + fb

    emb = emb_ref[0]
    s = jnp.dot(token_f, emb, preferred_element_type=jnp.float32)
    avg = s * (1.0 / (didx.astype(jnp.float32) + 1e-10))
    i64 = lax.broadcasted_iota(jnp.int32, (_L, _DC), 0)
    j64 = lax.broadcasted_iota(jnp.int32, (_L, _DC), 1)
    oh = (j64 == jnp.clip(i64 - didx, 0, _DC - 1)).astype(jnp.float32)
    dfg_rows = jnp.dot(oh, dfgo, preferred_element_type=jnp.float32)
    irow = lax.broadcasted_iota(jnp.int32, (_L, _HID), 0)
    rel = irow - didx
    selm = ((rel >= 0) & (rel < dlen)).astype(jnp.float32)
    emb = emb * (1.0 - selm) + ((1.0 - _ALPA) * avg + _ALPA * dfg_rows) * selm

    h = _ln(emb + prow_ref[0], lne[0:1], lne[1:2])
    qe = jnp.dot(h, wq_ref[...], preferred_element_type=jnp.float32) + bq
    ke = jnp.dot(h, wk_ref[...], preferred_element_type=jnp.float32) + bk
    ve = jnp.dot(h, wv_ref[...], preferred_element_type=jnp.float32) + bv
    addm = jnp.where(pos_b != 1, 0.0, -1e9)
    for hh in range(_NH):
        sl = slice(hh * _HD, (hh + 1) * _HD)
        s2 = lax.dot_general(qe[:, sl], ke[:, sl], (((1,), (1,)), ((), ())),
                             preferred_element_type=jnp.float32)
        s2 = s2 * (1.0 / math.sqrt(_HD)) + addm
        ctx_ref[:, sl] = jnp.dot(_softmax(s2), ve[:, sl],
                                 preferred_element_type=jnp.float32)
    attn = jnp.dot(ctx_ref[...], wo_ref[...],
                   preferred_element_type=jnp.float32) + bo
    h = _ln(h + attn, ln1[0:1], ln1[1:2])
    ff = jax.nn.gelu(jnp.dot(h, w1_ref[...],
                             preferred_element_type=jnp.float32) + b1_ref[...])
    ff = jnp.dot(ff, w2_ref[...], preferred_element_type=jnp.float32)
    ff = ff + _b2_from(biases)
    h = _ln(h + ff, ln2[0:1], ln2[1:2])
    pooled = jnp.tanh(
        jnp.dot(h[0:1, :], pw_ref[...], preferred_element_type=jnp.float32)
        + _pb_from(biases))
    out_ref[0] = pooled


def _b2_from(biases):
    return biases[5:6, _HID:2 * _HID]


def _pb_from(biases):
    return biases[6:7, _HID:2 * _HID]


def _run_fused(pos_all, inputs_emb, posrows, g, pos64, p):
    full = lambda shape: pl.BlockSpec(shape, lambda b: (0,) * len(shape))
    biases = jnp.zeros((8, _FF), jnp.float32)
    biases = biases.at[0, :_HID].set(p['Qb']).at[1, :_HID].set(p['Kb'])
    biases = biases.at[2, :_HID].set(p['Vb']).at[3, :_HID].set(p['ffb'])
    biases = biases.at[4, :_HID].set(p['enc_bq']).at[5, :_HID].set(p['enc_bk'])
    biases = biases.at[6, :_HID].set(p['enc_bv']).at[7, :_HID].set(p['enc_bo'])
    biases = biases.at[5, _HID:2 * _HID].set(p['enc_b2'])
    biases = biases.at[6, _HID:2 * _HID].set(p['pool_b'])
    lne = jnp.stack([p['ln_emb_g'], p['ln_emb_b']])
    ln1 = jnp.stack([p['ln1_g'], p['ln1_b']])
    ln2 = jnp.stack([p['ln2_g'], p['ln2_b']])
    return pl.pallas_call(
        _fused_body,
        grid=(_BS,),
        in_specs=[
            full((_BS, _L)),
            pl.BlockSpec((1, _L, _HID), lambda b: (b, 0, 0)),
            pl.BlockSpec((1, _L, _HID), lambda b: (b, 0, 0)),
            pl.BlockSpec((_DC, _HID), lambda b: (b, 0)),
            full((_DC, _HID)),
            full((_HID, _HID)), full((_HID, _HID)),
            full((_HID, _HID)), full((_HID, _HID)),
            full((2, _HID)),
            full((_HID, _HID)), full((_HID, _HID)),
            full((_HID, _HID)), full((_HID, _HID)),
            full((2, _HID)),
            full((_HID, _FF)), full((1, _FF)), full((_FF, _HID)),
            full((2, _HID)),
            full((_HID, _HID)),
            full((8, _FF)),
        ],
        out_specs=pl.BlockSpec((1, 1, _HID), lambda b: (b, 0, 0)),
        out_shape=jax.ShapeDtypeStruct((_BS, 1, _HID), jnp.float32),
        scratch_shapes=[pltpu.VMEM((_L, _HID), jnp.float32)],
    )(pos_all, inputs_emb, posrows, g, pos64,
      p['Qw'].T, p['Kw'].T, p['Vw'].T, p['ffw'].T,
      lne, p['enc_Wq'].T, p['enc_Wk'].T, p['enc_Wv'].T, p['enc_Wo'].T,
      ln1, p['enc_W1'].T, p['enc_b1'][None, :], p['enc_W2'].T,
      ln2, p['pool_W'].T, biases)


def kernel(code_inputs, attn_mask, position_idx, New_DFG_ids, params):
    p = params
    pos = position_idx.astype(jnp.int32)
    code = code_inputs.astype(jnp.int32)
    nd = New_DFG_ids.astype(jnp.int32)

    dfg_index = jnp.sum((pos >= 2).astype(jnp.int32), axis=1)
    idx = jnp.clip(dfg_index[:, None]
                   + jnp.arange(_DC, dtype=jnp.int32)[None, :], 0, _L - 1)
    fin_ids = jnp.take_along_axis(nd, idx[:, :, None], axis=1)

    widx = jnp.concatenate([code.reshape(-1), fin_ids.reshape(-1)])
    pidx = pos.reshape(-1)
    wrows, prows = _sc_gather(p['word_emb'], p['pos_emb'], widx, pidx)

    inputs_emb = wrows[:_BS * _L].reshape(_BS, _L, _HID)
    dfg_emb = wrows[_BS * _L:].reshape(_BS * _DC, _BSF, _HID)
    posrows = prows.reshape(_BS, _L, _HID)

    g = _run_gru(dfg_emb, p['gru_Wih'].T, p['gru_Whh'].T,
                 p['gru_bih'][None, :], p['gru_bhh'][None, :])

    pooled = _run_fused(pos, inputs_emb, posrows, g,
                        p['pos_emb'][:_DC], p)
    return pooled.reshape(_BS, _HID)


# speedup vs baseline: 1.8297x; 1.1211x over previous
import functools
import math

import jax
import jax.numpy as jnp
from jax import lax
from jax.experimental import pallas as pl
from jax.experimental.pallas import tpu as pltpu
from jax.experimental.pallas import tpu_sc as plsc

_HID = 768
_L = 512
_BS = 4
_BSF = 10
_NH = 12
_HD = 64
_FF = 3072
_ALPA = 0.6
_DC = 64

_NW = 32
_WTOT = _BS * _L + _BS * _DC * _BSF
_WPW = _WTOT // _NW
_PTOT = _BS * _L
_PPW = _PTOT // _NW


def _sc_gather(wemb, pemb, widx, pidx):
    mesh = plsc.VectorSubcoreMesh(core_axis_name="c", subcore_axis_name="s")

    @functools.partial(
        pl.kernel,
        out_type=[
            jax.ShapeDtypeStruct((_WTOT, _HID), jnp.float32),
            jax.ShapeDtypeStruct((_PTOT, _HID), jnp.float32),
        ],
        mesh=mesh,
        scratch_types=[
            pltpu.VMEM((_WPW,), jnp.int32),
            pltpu.VMEM((_PPW,), jnp.int32),
            pltpu.VMEM((_WPW, _HID), jnp.float32),
            pltpu.SemaphoreType.DMA,
        ],
    )
    def k(wemb_h, pemb_h, widx_h, pidx_h, wout_h, pout_h,
          widx_v, pidx_v, rows_v, sem):
        wid = lax.axis_index("s") * 2 + lax.axis_index("c")
        wb = wid * _WPW
        pltpu.sync_copy(widx_h.at[pl.ds(wb, _WPW)], widx_v)
        pltpu.async_copy(wemb_h.at[widx_v], rows_v, sem).wait()
        pltpu.sync_copy(rows_v, wout_h.at[pl.ds(wb, _WPW)])
        pb = wid * _PPW
        pltpu.sync_copy(pidx_h.at[pl.ds(pb, _PPW)], pidx_v)
        pltpu.async_copy(pemb_h.at[pidx_v], rows_v.at[pl.ds(0, _PPW)], sem).wait()
        pltpu.sync_copy(rows_v.at[pl.ds(0, _PPW)], pout_h.at[pl.ds(pb, _PPW)])

    return k(wemb, pemb, widx, pidx)


def _mmt(x, w):
    return lax.dot_general(x, w, (((1,), (1,)), ((), ())),
                           preferred_element_type=jnp.float32)

def _gru_body(x_ref, wih_ref, whh_ref, bih_ref, bhh_ref, out_ref):
    n = x_ref.shape[0]
    wih = wih_ref[...]
    whh = whh_ref[...]
    bih = bih_ref[...]
    bhh = bhh_ref[...]
    h = jnp.zeros((n, _HID), jnp.float32)
    for t in range(_BSF):
        x = x_ref[:, t, :]
        gi = _mmt(x, wih) + bih
        gh = _mmt(h, whh) + bhh
        r = jax.nn.sigmoid(gi[:, :_HID] + gh[:, :_HID])
        z = jax.nn.sigmoid(gi[:, _HID:2 * _HID] + gh[:, _HID:2 * _HID])
        nn = jnp.tanh(gi[:, 2 * _HID:] + r * gh[:, 2 * _HID:])
        h = (1.0 - z) * nn + z * h
    out_ref[...] = h


def _run_gru(dfg_emb, wih_t, whh_t, bih, bhh):
    n = dfg_emb.shape[0]
    return pl.pallas_call(
        _gru_body,
        out_shape=jax.ShapeDtypeStruct((n, _HID), jnp.float32),
    )(dfg_emb, wih_t, whh_t, bih, bhh)


def _ln(x, g, b):
    m = jnp.mean(x, axis=-1, keepdims=True)
    v = jnp.mean((x - m) * (x - m), axis=-1, keepdims=True)
    return (x - m) / jnp.sqrt(v + 1e-5) * g + b


def _softmax(x):
    m = jnp.max(x, axis=-1, keepdims=True)
    e = jnp.exp(x - m)
    return e / jnp.sum(e, axis=-1, keepdims=True)


def _fused_body(pos_all_ref, emb_ref, prow_ref, g_ref, pos64_ref,
                qw_ref, kw_ref, vw_ref, fw_ref,
                lne_ref, wq_ref, wk_ref, wv_ref, wo_ref,
                ln1_ref, w1_ref, b1_ref, w2_ref,
                ln2_ref, pw_ref, bias_ref,
                out_ref, ctx_ref):
    b = pl.program_id(0)
    pos_all = pos_all_ref[...]
    dfg_len_all = jnp.sum((pos_all == 0).astype(jnp.int32), axis=1,
                          keepdims=True)
    dmax = jnp.max(dfg_len_all)
    pos_b = pos_all_ref[pl.ds(b, 1), :]
    token_f = (pos_b >= 2).astype(jnp.float32)
    didx = jnp.sum((pos_b >= 2).astype(jnp.int32))
    dlen = jnp.sum((pos_b == 0).astype(jnp.int32))

    biases = bias_ref[...]
    qb, kb, vb, fb = (biases[0:1, :_HID], biases[1:2, :_HID],
                      biases[2:3, :_HID], biases[3:4, :_HID])
    bq, bk, bv, bo = (biases[4:5, :_HID], biases[5:6, :_HID],
                      biases[6:7, :_HID], biases[7:8, :_HID])
    lne = lne_ref[...]
    ln1 = ln1_ref[...]
    ln2 = ln2_ref[...]

    dfg = g_ref[...] + pos64_ref[...]
    q = _mmt(dfg, qw_ref[...]) + qb
    k = _mmt(dfg, kw_ref[...]) + kb
    v = _mmt(dfg, vw_ref[...]) + vb
    sc = lax.dot_general(q, k, (((1,), (1,)), ((), ())),
                         preferred_element_type=jnp.float32)
    sc = sc * (1.0 / math.sqrt(_HID))
    col = lax.broadcasted_iota(jnp.int32, (_DC, _DC), 1)
    sc = jnp.where(col < dmax, sc, -jnp.inf)
    dfgo = jnp.dot(_softmax(sc), v, preferred_element_type=jnp.float32)
    dfgo = _mmt(dfgo, fw_ref[...]) + fb

    emb = emb_ref[0]
    s = jnp.dot(token_f, emb, preferred_element_type=jnp.float32)
    avg = s * (1.0 / (didx.astype(jnp.float32) + 1e-10))
    i64 = lax.broadcasted_iota(jnp.int32, (_L, _DC), 0)
    j64 = lax.broadcasted_iota(jnp.int32, (_L, _DC), 1)
    oh = (j64 == jnp.clip(i64 - didx, 0, _DC - 1)).astype(jnp.float32)
    dfg_rows = jnp.dot(oh, dfgo, preferred_element_type=jnp.float32)
    irow = lax.broadcasted_iota(jnp.int32, (_L, _HID), 0)
    rel = irow - didx
    selm = ((rel >= 0) & (rel < dlen)).astype(jnp.float32)
    emb = emb * (1.0 - selm) + ((1.0 - _ALPA) * avg + _ALPA * dfg_rows) * selm

    h = _ln(emb + prow_ref[0], lne[0:1], lne[1:2])
    qe = _mmt(h, wq_ref[...]) + bq
    ke = _mmt(h, wk_ref[...]) + bk
    ve = _mmt(h, wv_ref[...]) + bv
    addm = jnp.where(pos_b != 1, 0.0, -1e9)
    for hh in range(_NH):
        sl = slice(hh * _HD, (hh + 1) * _HD)
        s2 = lax.dot_general(qe[:, sl], ke[:, sl], (((1,), (1,)), ((), ())),
                             preferred_element_type=jnp.float32)
        s2 = s2 * (1.0 / math.sqrt(_HD)) + addm
        ctx_ref[:, sl] = jnp.dot(_softmax(s2), ve[:, sl],
                                 preferred_element_type=jnp.float32)
    attn = _mmt(ctx_ref[...], wo_ref[...]) + bo
    h = _ln(h + attn, ln1[0:1], ln1[1:2])
    ff = jax.nn.gelu(_mmt(h, w1_ref[...]) + b1_ref[...])
    ff = _mmt(ff, w2_ref[...])
    ff = ff + _b2_from(biases)
    h = _ln(h + ff, ln2[0:1], ln2[1:2])
    pooled = jnp.tanh(_mmt(h[0:1, :], pw_ref[...]) + _pb_from(biases))
    out_ref[0] = pooled


def _b2_from(biases):
    return biases[5:6, _HID:2 * _HID]


def _pb_from(biases):
    return biases[6:7, _HID:2 * _HID]


def _run_fused(pos_all, inputs_emb, posrows, g, pos64, p):
    full = lambda shape: pl.BlockSpec(shape, lambda b: (0,) * len(shape))
    biases = jnp.zeros((8, _FF), jnp.float32)
    biases = biases.at[0, :_HID].set(p['Qb']).at[1, :_HID].set(p['Kb'])
    biases = biases.at[2, :_HID].set(p['Vb']).at[3, :_HID].set(p['ffb'])
    biases = biases.at[4, :_HID].set(p['enc_bq']).at[5, :_HID].set(p['enc_bk'])
    biases = biases.at[6, :_HID].set(p['enc_bv']).at[7, :_HID].set(p['enc_bo'])
    biases = biases.at[5, _HID:2 * _HID].set(p['enc_b2'])
    biases = biases.at[6, _HID:2 * _HID].set(p['pool_b'])
    lne = jnp.stack([p['ln_emb_g'], p['ln_emb_b']])
    ln1 = jnp.stack([p['ln1_g'], p['ln1_b']])
    ln2 = jnp.stack([p['ln2_g'], p['ln2_b']])
    return pl.pallas_call(
        _fused_body,
        grid=(_BS,),
        in_specs=[
            full((_BS, _L)),
            pl.BlockSpec((1, _L, _HID), lambda b: (b, 0, 0)),
            pl.BlockSpec((1, _L, _HID), lambda b: (b, 0, 0)),
            pl.BlockSpec((_DC, _HID), lambda b: (b, 0)),
            full((_DC, _HID)),
            full((_HID, _HID)), full((_HID, _HID)),
            full((_HID, _HID)), full((_HID, _HID)),
            full((2, _HID)),
            full((_HID, _HID)), full((_HID, _HID)),
            full((_HID, _HID)), full((_HID, _HID)),
            full((2, _HID)),
            full((_FF, _HID)), full((1, _FF)), full((_HID, _FF)),
            full((2, _HID)),
            full((_HID, _HID)),
            full((8, _FF)),
        ],
        out_specs=pl.BlockSpec((1, 1, _HID), lambda b: (b, 0, 0)),
        out_shape=jax.ShapeDtypeStruct((_BS, 1, _HID), jnp.float32),
        scratch_shapes=[pltpu.VMEM((_L, _HID), jnp.float32)],
    )(pos_all, inputs_emb, posrows, g, pos64,
      p['Qw'], p['Kw'], p['Vw'], p['ffw'],
      lne, p['enc_Wq'], p['enc_Wk'], p['enc_Wv'], p['enc_Wo'],
      ln1, p['enc_W1'], p['enc_b1'][None, :], p['enc_W2'],
      ln2, p['pool_W'], biases)


def kernel(code_inputs, attn_mask, position_idx, New_DFG_ids, params):
    p = params
    pos = position_idx.astype(jnp.int32)
    code = code_inputs.astype(jnp.int32)
    nd = New_DFG_ids.astype(jnp.int32)

    dfg_index = jnp.sum((pos >= 2).astype(jnp.int32), axis=1)
    idx = jnp.clip(dfg_index[:, None]
                   + jnp.arange(_DC, dtype=jnp.int32)[None, :], 0, _L - 1)
    fin_ids = jnp.take_along_axis(nd, idx[:, :, None], axis=1)

    widx = jnp.concatenate([code.reshape(-1), fin_ids.reshape(-1)])
    pidx = pos.reshape(-1)
    wrows, prows = _sc_gather(p['word_emb'], p['pos_emb'], widx, pidx)

    inputs_emb = wrows[:_BS * _L].reshape(_BS, _L, _HID)
    dfg_emb = wrows[_BS * _L:].reshape(_BS * _DC, _BSF, _HID)
    posrows = prows.reshape(_BS, _L, _HID)

    g = _run_gru(dfg_emb, p['gru_Wih'], p['gru_Whh'],
                 p['gru_bih'][None, :], p['gru_bhh'][None, :])

    pooled = _run_fused(pos, inputs_emb, posrows, g,
                        p['pos_emb'][:_DC], p)
    return pooled.reshape(_BS, _HID)
